# baseline (speedup 1.0000x reference)
import jax
import jax.numpy as jnp
from jax.experimental import pallas as pl

_LEVELS = 4
_K = 1024
_D = 256
_N = 16384
_T = 512


def _level_body(r_ref, qsum_ref, cb_ref, rnorm_ref, cnorm_ref,
                idx_ref, rout_ref, qout_ref):
    r = r_ref[...]
    cb = cb_ref[...]
    m = jax.lax.dot_general(r, cb, (((1,), (1,)), ((), ())),
                            preferred_element_type=jnp.float32)
    d2 = (rnorm_ref[...] - 2.0 * m) + cnorm_ref[...]
    dist = jnp.sqrt(jnp.maximum(d2, 0.0))
    idx = jnp.argmin(dist, axis=1).astype(jnp.int32)
    iota_k = jax.lax.broadcasted_iota(jnp.int32, (1, _K), 1)
    onehot = (iota_k == idx[:, None]).astype(jnp.float32)
    q = jax.lax.dot_general(onehot, cb, (((1,), (0,)), ((), ())),
                            preferred_element_type=jnp.float32,
                            precision=jax.lax.Precision.HIGHEST)
    idx_ref[...] = idx.reshape(_T, 1)
    rout_ref[...] = r - q
    qout_ref[...] = qsum_ref[...] + q


def _level_call(r, qsum, cb, rnorm, cnorm):
    nb = _N // _T
    return pl.pallas_call(
        _level_body,
        grid=(nb,),
        in_specs=[
            pl.BlockSpec((_T, _D), lambda i: (i, 0)),
            pl.BlockSpec((_T, _D), lambda i: (i, 0)),
            pl.BlockSpec((_K, _D), lambda i: (0, 0)),
            pl.BlockSpec((_T, 1), lambda i: (i, 0)),
            pl.BlockSpec((1, _K), lambda i: (0, 0)),
        ],
        out_specs=[
            pl.BlockSpec((_T, 1), lambda i: (i, 0)),
            pl.BlockSpec((_T, _D), lambda i: (i, 0)),
            pl.BlockSpec((_T, _D), lambda i: (i, 0)),
        ],
        out_shape=[
            jax.ShapeDtypeStruct((_N, 1), jnp.int32),
            jax.ShapeDtypeStruct((_N, _D), jnp.float32),
            jax.ShapeDtypeStruct((_N, _D), jnp.float32),
        ],
    )(r, qsum, cb, rnorm, cnorm)


def kernel(z, codebooks):
    r = z
    qsum = jnp.zeros_like(z)
    idxs = []
    for l in range(_LEVELS):
        cb = codebooks[l]
        rnorm = jnp.sum(r * r, axis=1, keepdims=True)
        cnorm = jnp.sum(cb * cb, axis=1).reshape(1, _K)
        idx, r, qsum = _level_call(r, qsum, cb, rnorm, cnorm)
        idxs.append(idx[:, 0])
    return qsum, jnp.stack(idxs, axis=0)


# speedup vs baseline: 1.0672x; 1.0672x over previous
import functools

import jax
import jax.numpy as jnp
from jax.experimental import pallas as pl
from jax.experimental.pallas import tpu as pltpu
from jax.experimental.pallas import tpu_sc as plsc

_LEVELS = 4
_K = 1024
_D = 256
_N = 16384
_T = 512

_SC_INFO = plsc.get_sparse_core_info()
_NC, _NS = _SC_INFO.num_cores, _SC_INFO.num_subcores
_NW = _NC * _NS
_CHUNK = 128
_B_PER_W = _N // _NW
_N_CHUNKS = _B_PER_W // _CHUNK


def _argmin_body(r_ref, cb_ref, rnorm_ref, cnorm_ref, idx_ref):
    r = r_ref[...]
    cb = cb_ref[...]
    m = jax.lax.dot_general(r, cb, (((1,), (1,)), ((), ())),
                            preferred_element_type=jnp.float32)
    d2 = (rnorm_ref[...] - 2.0 * m) + cnorm_ref[...]
    dist = jnp.sqrt(jnp.maximum(d2, 0.0))
    idx_ref[...] = jnp.argmin(dist, axis=1).astype(jnp.int32).reshape(_T, 1)


def _tc_argmin(r, cb, rnorm, cnorm):
    nb = _N // _T
    return pl.pallas_call(
        _argmin_body,
        grid=(nb,),
        in_specs=[
            pl.BlockSpec((_T, _D), lambda i: (i, 0)),
            pl.BlockSpec((_K, _D), lambda i: (0, 0)),
            pl.BlockSpec((_T, 1), lambda i: (i, 0)),
            pl.BlockSpec((1, _K), lambda i: (0, 0)),
        ],
        out_specs=pl.BlockSpec((_T, 1), lambda i: (i, 0)),
        out_shape=jax.ShapeDtypeStruct((_N, 1), jnp.int32),
    )(r, cb, rnorm, cnorm)


def _sc_gather(cb, idx):
    @functools.partial(
        pl.kernel,
        mesh=plsc.VectorSubcoreMesh(core_axis_name="c", subcore_axis_name="s"),
        out_type=jax.ShapeDtypeStruct((_N, _D), jnp.float32),
        scratch_types=[
            pltpu.VMEM((_CHUNK,), jnp.int32),
            pltpu.VMEM((_CHUNK, _D), jnp.float32),
            pltpu.SemaphoreType.DMA,
        ],
    )
    def k(cb_hbm, idx_hbm, out_hbm, idx_v, rows_v, sem):
        wid = jax.lax.axis_index("s") * _NC + jax.lax.axis_index("c")
        for c in range(_N_CHUNKS):
            base = wid * _B_PER_W + c * _CHUNK
            pltpu.sync_copy(idx_hbm.at[pl.ds(base, _CHUNK)], idx_v)
            pltpu.async_copy(cb_hbm.at[idx_v], rows_v, sem).wait()
            pltpu.sync_copy(rows_v, out_hbm.at[pl.ds(base, _CHUNK)])

    return k(cb, idx)


def _qsum_body(q0_ref, q1_ref, q2_ref, q3_ref, out_ref):
    out_ref[...] = ((q0_ref[...] + q1_ref[...]) + q2_ref[...]) + q3_ref[...]


def _tc_qsum(q0, q1, q2, q3):
    nb = _N // _T
    spec = pl.BlockSpec((_T, _D), lambda i: (i, 0))
    return pl.pallas_call(
        _qsum_body,
        grid=(nb,),
        in_specs=[spec, spec, spec, spec],
        out_specs=spec,
        out_shape=jax.ShapeDtypeStruct((_N, _D), jnp.float32),
    )(q0, q1, q2, q3)


def kernel(z, codebooks):
    r = z
    qs = []
    idxs = []
    for l in range(_LEVELS):
        cb = codebooks[l]
        rnorm = jnp.sum(r * r, axis=1, keepdims=True)
        cnorm = jnp.sum(cb * cb, axis=1).reshape(1, _K)
        idx = _tc_argmin(r, cb, rnorm, cnorm)
        q = _sc_gather(cb, idx.reshape(_N))
        qs.append(q)
        idxs.append(idx[:, 0])
        if l < _LEVELS - 1:
            r = r - q
    return _tc_qsum(*qs), jnp.stack(idxs, axis=0)


# speedup vs baseline: 1.0840x; 1.0157x over previous
import functools

import jax
import jax.numpy as jnp
from jax.experimental import pallas as pl
from jax.experimental.pallas import tpu as pltpu
from jax.experimental.pallas import tpu_sc as plsc

_LEVELS = 4
_K = 1024
_D = 256
_N = 16384
_T = 512
_SLABS = 2
_H = _N // _SLABS

_SC_INFO = plsc.get_sparse_core_info()
_NC, _NS = _SC_INFO.num_cores, _SC_INFO.num_subcores
_NW = _NC * _NS
_CHUNK = 128


def _argmin_body(r_ref, cb_ref, rnorm_ref, cnorm_ref, idx_ref):
    r = r_ref[...]
    cb = cb_ref[...]
    m = jax.lax.dot_general(r, cb, (((1,), (1,)), ((), ())),
                            preferred_element_type=jnp.float32)
    d2 = (rnorm_ref[...] - 2.0 * m) + cnorm_ref[...]
    dist = jnp.sqrt(jnp.maximum(d2, 0.0))
    idx_ref[...] = jnp.argmin(dist, axis=1).astype(jnp.int32).reshape(_T, 1)


def _tc_argmin(r, cb, rnorm, cnorm):
    n = r.shape[0]
    return pl.pallas_call(
        _argmin_body,
        grid=(n // _T,),
        in_specs=[
            pl.BlockSpec((_T, _D), lambda i: (i, 0)),
            pl.BlockSpec((_K, _D), lambda i: (0, 0)),
            pl.BlockSpec((_T, 1), lambda i: (i, 0)),
            pl.BlockSpec((1, _K), lambda i: (0, 0)),
        ],
        out_specs=pl.BlockSpec((_T, 1), lambda i: (i, 0)),
        out_shape=jax.ShapeDtypeStruct((n, 1), jnp.int32),
    )(r, cb, rnorm, cnorm)


def _sc_gather(cb, idx):
    n = idx.shape[0]
    b_per_w = n // _NW
    n_chunks = b_per_w // _CHUNK

    @functools.partial(
        pl.kernel,
        mesh=plsc.VectorSubcoreMesh(core_axis_name="c", subcore_axis_name="s"),
        out_type=jax.ShapeDtypeStruct((n, _D), jnp.float32),
        scratch_types=[
            pltpu.VMEM((_CHUNK,), jnp.int32),
            pltpu.VMEM((_CHUNK, _D), jnp.float32),
            pltpu.SemaphoreType.DMA,
        ],
    )
    def k(cb_hbm, idx_hbm, out_hbm, idx_v, rows_v, sem):
        wid = jax.lax.axis_index("s") * _NC + jax.lax.axis_index("c")
        for c in range(n_chunks):
            base = wid * b_per_w + c * _CHUNK
            pltpu.sync_copy(idx_hbm.at[pl.ds(base, _CHUNK)], idx_v)
            pltpu.async_copy(cb_hbm.at[idx_v], rows_v, sem).wait()
            pltpu.sync_copy(rows_v, out_hbm.at[pl.ds(base, _CHUNK)])

    return k(cb, idx)


def _qsum_body(q0_ref, q1_ref, q2_ref, q3_ref, out_ref):
    out_ref[...] = ((q0_ref[...] + q1_ref[...]) + q2_ref[...]) + q3_ref[...]


def _tc_qsum(q0, q1, q2, q3):
    n = q0.shape[0]
    spec = pl.BlockSpec((_T, _D), lambda i: (i, 0))
    return pl.pallas_call(
        _qsum_body,
        grid=(n // _T,),
        in_specs=[spec, spec, spec, spec],
        out_specs=spec,
        out_shape=jax.ShapeDtypeStruct((n, _D), jnp.float32),
    )(q0, q1, q2, q3)


def kernel(z, codebooks):
    cbs = [codebooks[l] for l in range(_LEVELS)]
    cnorms = [jnp.sum(cb * cb, axis=1).reshape(1, _K) for cb in cbs]
    qsum_slabs = []
    idx_slabs = []
    for s in range(_SLABS):
        r = z[s * _H:(s + 1) * _H]
        qs = []
        idxs = []
        for l in range(_LEVELS):
            rnorm = jnp.sum(r * r, axis=1, keepdims=True)
            idx = _tc_argmin(r, cbs[l], rnorm, cnorms[l])
            q = _sc_gather(cbs[l], idx.reshape(_H))
            qs.append(q)
            idxs.append(idx[:, 0])
            if l < _LEVELS - 1:
                r = r - q
        qsum_slabs.append(_tc_qsum(*qs))
        idx_slabs.append(jnp.stack(idxs, axis=0))
    return (jnp.concatenate(qsum_slabs, axis=0),
            jnp.concatenate(idx_slabs, axis=1))
